# Initial kernel scaffold; baseline (speedup 1.0000x reference)
import jax
import jax.numpy as jnp
from jax.experimental import pallas as pl


def kernel(node_feat, segment_ids):
    raise NotImplementedError("write your pallas kernel here")


# speedup vs baseline: 5.3667x; 5.3667x over previous
import functools

import jax
import jax.numpy as jnp
from jax import lax
from jax.experimental import pallas as pl
from jax.experimental.pallas import tpu as pltpu
from jax.experimental.pallas import tpu_sc as plsc

N_ROWS = 100000
D = 128
NSEG = 64
GARBAGE = NSEG
ACC_ROWS = 72
CHUNK = 128
CW = 128
NC, NS = 2, 16
NW = NC * NS
NCHUNKS = -(-N_ROWS // CHUNK)
LAST = NCHUNKS - 1
OVERLAP = NCHUNKS * CHUNK - N_ROWS
MAX_ITERS = -(-NCHUNKS // NW)
CHUNKS_PER_W, EXTRA_W = divmod(NCHUNKS, NW)


def _sc_segment_sums(node_feat, ids_aug, ones, zrow, zcnt):
    mesh = plsc.VectorSubcoreMesh(
        core_axis_name="c", subcore_axis_name="s",
        num_cores=NC, num_subcores=NS)

    @functools.partial(
        pl.kernel,
        out_type=(
            jax.ShapeDtypeStruct((NC, ACC_ROWS, D), jnp.float32),
            jax.ShapeDtypeStruct((NC, ACC_ROWS, CW), jnp.float32),
        ),
        mesh=mesh,
        scratch_types=[
            pltpu.VMEM((2, CHUNK, D), jnp.float32),
            pltpu.VMEM((2, CHUNK), jnp.int32),
            pltpu.VMEM((1, CHUNK, CW), jnp.float32),
            pltpu.VMEM((ACC_ROWS, D), jnp.float32),
            pltpu.VMEM((ACC_ROWS, CW), jnp.float32),
            pltpu.VMEM_SHARED((ACC_ROWS, D), jnp.float32),
            pltpu.VMEM_SHARED((ACC_ROWS, CW), jnp.float32),
            pltpu.SemaphoreType.DMA,
            pltpu.SemaphoreType.DMA,
        ],
    )
    def seg_sum(feat_hbm, ids_hbm, ones_hbm, zrow_hbm, zcnt_hbm,
                sums_hbm, cnts_hbm,
                fbuf, ibuf, ones_v, zbuf, cbuf, acc_sh, cnt_sh, sem0, sem1):
        ci = lax.axis_index("c")
        si = lax.axis_index("s")
        wid = si * NC + ci
        start = wid * CHUNKS_PER_W + jnp.minimum(wid, EXTRA_W)
        n_w = CHUNKS_PER_W + jnp.where(wid < EXTRA_W, 1, 0)
        sems = (sem0, sem1)

        @pl.when(si == 0)
        def _():
            pltpu.sync_copy(zrow_hbm, zbuf)
            pltpu.sync_copy(zbuf, acc_sh)
            pltpu.sync_copy(zcnt_hbm, cbuf)
            pltpu.sync_copy(cbuf, cnt_sh)
        pltpu.sync_copy(ones_hbm, ones_v)
        plsc.subcore_barrier()

        def issue(i, b):
            c = start + i
            feat_base = pl.multiple_of(
                jnp.where(c == LAST, N_ROWS - CHUNK, c * CHUNK), 8)
            idx_base = pl.multiple_of(
                jnp.where(c == LAST, N_ROWS, c * CHUNK), 8)
            pltpu.async_copy(
                feat_hbm.at[pl.ds(feat_base, CHUNK)], fbuf.at[b], sems[b])
            pltpu.async_copy(
                ids_hbm.at[pl.ds(idx_base, CHUNK)], ibuf.at[b], sems[b])

        def wait(b):
            pltpu.make_async_copy(
                feat_hbm.at[pl.ds(0, CHUNK)], fbuf.at[b], sems[b]).wait()
            pltpu.make_async_copy(
                ids_hbm.at[pl.ds(0, CHUNK)], ibuf.at[b], sems[b]).wait()

        @pl.when(0 < n_w)
        def _():
            issue(0, 0)

        @pl.when(1 < n_w)
        def _():
            issue(1, 1)

        for i in range(MAX_ITERS):
            b = i & 1

            @pl.when(i < n_w)
            def _(i=i, b=b):
                wait(b)
                pltpu.sync_copy(fbuf.at[b], acc_sh.at[ibuf.at[b]], add=True)
                pltpu.sync_copy(ones_v.at[0], cnt_sh.at[ibuf.at[b]], add=True)

                @pl.when(i + 2 < n_w)
                def _():
                    issue(i + 2, b)

        plsc.subcore_barrier()

        @pl.when(si == 0)
        def _():
            pltpu.sync_copy(acc_sh, zbuf)
            pltpu.sync_copy(zbuf, sums_hbm.at[ci])
            pltpu.sync_copy(cnt_sh, cbuf)
            pltpu.sync_copy(cbuf, cnts_hbm.at[ci])

    return seg_sum(node_feat, ids_aug, ones, zrow, zcnt)


def _combine_body(s_ref, c_ref, o_ref):
    s = s_ref[0, :NSEG, :] + s_ref[1, :NSEG, :]
    c = c_ref[0, :NSEG, :] + c_ref[1, :NSEG, :]
    o_ref[...] = s / c


def _combine(sums, cnts):
    return pl.pallas_call(
        _combine_body,
        out_shape=jax.ShapeDtypeStruct((NSEG, D), jnp.float32),
    )(sums, cnts)


@jax.jit
def kernel(node_feat, segment_ids):
    ids32 = segment_ids.astype(jnp.int32)
    idx_last = jnp.concatenate(
        [jnp.full((OVERLAP,), GARBAGE, jnp.int32), ids32[N_ROWS - CHUNK + OVERLAP:]])
    ids_aug = jnp.concatenate([ids32, idx_last])
    ones = jnp.ones((1, CHUNK, CW), jnp.float32)
    zrow = jnp.zeros((ACC_ROWS, D), jnp.float32)
    zcnt = jnp.zeros((ACC_ROWS, CW), jnp.float32)
    sums, cnts = _sc_segment_sums(node_feat, ids_aug, ones, zrow, zcnt)
    return _combine(sums, cnts)


# speedup vs baseline: 8.4922x; 1.5824x over previous
import functools

import jax
import jax.numpy as jnp
from jax import lax
from jax.experimental import pallas as pl
from jax.experimental.pallas import tpu as pltpu
from jax.experimental.pallas import tpu_sc as plsc

N_ROWS = 100000
D = 128
NSEG = 64
GARBAGE = NSEG
ACC_ROWS = 72
CHUNK = 128
NC, NS = 2, 16
NW = NC * NS
NCHUNKS = -(-N_ROWS // CHUNK)
LAST = NCHUNKS - 1
OVERLAP = NCHUNKS * CHUNK - N_ROWS
MAX_ITERS = -(-NCHUNKS // NW)
CHUNKS_PER_W, EXTRA_W = divmod(NCHUNKS, NW)
HIST_BLK = 8
R_TC = -(-N_ROWS // (HIST_BLK * D)) * HIST_BLK


def _sc_segment_sums(node_feat, ids_aug, zrow):
    mesh = plsc.VectorSubcoreMesh(
        core_axis_name="c", subcore_axis_name="s",
        num_cores=NC, num_subcores=NS)

    @functools.partial(
        pl.kernel,
        out_type=jax.ShapeDtypeStruct((NC, ACC_ROWS, D), jnp.float32),
        mesh=mesh,
        scratch_types=[
            pltpu.VMEM((2, CHUNK, D), jnp.float32),
            pltpu.VMEM((2, CHUNK), jnp.int32),
            pltpu.VMEM((ACC_ROWS, D), jnp.float32),
            pltpu.VMEM_SHARED((ACC_ROWS, D), jnp.float32),
            pltpu.SemaphoreType.DMA,
            pltpu.SemaphoreType.DMA,
        ],
    )
    def seg_sum(feat_hbm, ids_hbm, zrow_hbm, sums_hbm,
                fbuf, ibuf, zbuf, acc_sh, sem0, sem1):
        ci = lax.axis_index("c")
        si = lax.axis_index("s")
        wid = si * NC + ci
        start = wid * CHUNKS_PER_W + jnp.minimum(wid, EXTRA_W)
        n_w = CHUNKS_PER_W + jnp.where(wid < EXTRA_W, 1, 0)
        sems = (sem0, sem1)

        @pl.when(si == 0)
        def _():
            pltpu.sync_copy(zrow_hbm, zbuf)
            pltpu.sync_copy(zbuf, acc_sh)
        plsc.subcore_barrier()

        def issue(i, b):
            c = start + i
            feat_base = pl.multiple_of(
                jnp.where(c == LAST, N_ROWS - CHUNK, c * CHUNK), 8)
            idx_base = pl.multiple_of(
                jnp.where(c == LAST, N_ROWS, c * CHUNK), 8)
            pltpu.async_copy(
                feat_hbm.at[pl.ds(feat_base, CHUNK)], fbuf.at[b], sems[b])
            pltpu.async_copy(
                ids_hbm.at[pl.ds(idx_base, CHUNK)], ibuf.at[b], sems[b])

        def wait(b):
            pltpu.make_async_copy(
                feat_hbm.at[pl.ds(0, CHUNK)], fbuf.at[b], sems[b]).wait()
            pltpu.make_async_copy(
                ids_hbm.at[pl.ds(0, CHUNK)], ibuf.at[b], sems[b]).wait()

        @pl.when(0 < n_w)
        def _():
            issue(0, 0)

        @pl.when(1 < n_w)
        def _():
            issue(1, 1)

        for i in range(MAX_ITERS):
            b = i & 1

            @pl.when(i < n_w)
            def _(i=i, b=b):
                wait(b)
                pltpu.sync_copy(fbuf.at[b], acc_sh.at[ibuf.at[b]], add=True)

                @pl.when(i + 2 < n_w)
                def _():
                    issue(i + 2, b)

        plsc.subcore_barrier()

        @pl.when(si == 0)
        def _():
            pltpu.sync_copy(acc_sh, zbuf)
            pltpu.sync_copy(zbuf, sums_hbm.at[ci])

    return seg_sum(node_feat, ids_aug, zrow)


def _hist_body(i_ref, o_ref):
    r = pl.program_id(0)

    @pl.when(r == 0)
    def _():
        o_ref[...] = jnp.zeros((NSEG, D), jnp.float32)

    seg = lax.broadcasted_iota(jnp.int32, (NSEG, D), 0)
    acc = o_ref[...]
    for j in range(HIST_BLK):
        row = i_ref[j:j + 1, :]
        acc = acc + (jnp.broadcast_to(row, (NSEG, D)) == seg).astype(jnp.float32)
    o_ref[...] = acc


def _tc_histogram(ids_2d):
    return pl.pallas_call(
        _hist_body,
        grid=(R_TC // HIST_BLK,),
        in_specs=[pl.BlockSpec((HIST_BLK, D), lambda r: (r, 0))],
        out_specs=pl.BlockSpec((NSEG, D), lambda r: (0, 0)),
        out_shape=jax.ShapeDtypeStruct((NSEG, D), jnp.float32),
    )(ids_2d)


def _combine_body(s_ref, h_ref, o_ref):
    s = s_ref[0, :NSEG, :] + s_ref[1, :NSEG, :]
    cnt = jnp.sum(h_ref[...], axis=1, keepdims=True)
    o_ref[...] = s / cnt


def _combine(sums, hist):
    return pl.pallas_call(
        _combine_body,
        out_shape=jax.ShapeDtypeStruct((NSEG, D), jnp.float32),
    )(sums, hist)


@jax.jit
def kernel(node_feat, segment_ids):
    ids32 = segment_ids.astype(jnp.int32)
    idx_last = jnp.concatenate(
        [jnp.full((OVERLAP,), GARBAGE, jnp.int32),
         ids32[N_ROWS - CHUNK + OVERLAP:]])
    ids_aug = jnp.concatenate([ids32, idx_last])
    ids_2d = jnp.concatenate(
        [ids32, jnp.full((R_TC * D - N_ROWS,), GARBAGE, jnp.int32)]
    ).reshape(R_TC, D)
    zrow = jnp.zeros((ACC_ROWS, D), jnp.float32)
    hist = _tc_histogram(ids_2d)
    sums = _sc_segment_sums(node_feat, ids_aug, zrow)
    return _combine(sums, hist)
